# baseline (speedup 1.0000x reference)
import functools

import jax
import jax.numpy as jnp
from jax import lax
from jax.experimental import pallas as pl
from jax.experimental.pallas import tpu as pltpu
from jax.experimental.pallas import tpu_sc as plsc

_LANES = 16


@functools.lru_cache(maxsize=None)
def _make_edge_cols_kernel(total: int, bsz: int):
    info = plsc.get_sparse_core_info()
    nw = info.num_cores * info.num_subcores
    n_edges = 2 * (total - bsz)
    assert n_edges % _LANES == 0
    ch = -(-n_edges // nw)
    ch = -(-ch // _LANES) * _LANES
    last = n_edges - (nw - 1) * ch
    assert 0 < last <= ch and last % _LANES == 0 and ch % 8 == 0
    rows = ch // _LANES

    mesh = plsc.VectorSubcoreMesh(core_axis_name="c", subcore_axis_name="s")

    @functools.partial(
        pl.kernel,
        mesh=mesh,
        out_type=(
            jax.ShapeDtypeStruct((n_edges,), jnp.int32),
            jax.ShapeDtypeStruct((n_edges,), jnp.int32),
        ),
        scratch_types=[
            pltpu.VMEM((_LANES,), jnp.int32),
            pltpu.VMEM((ch,), jnp.int32),
            pltpu.VMEM((ch,), jnp.int32),
        ],
    )
    def k(rl_hbm, c0_hbm, c1_hbm, rl_v, buf0, buf1):
        wid = lax.axis_index("s") * info.num_cores + lax.axis_index("c")

        pltpu.sync_copy(rl_hbm, rl_v)
        rl_vec = rl_v[...]
        thr = []
        run = None
        for t in range(bsz - 1):
            rl_b = rl_vec.at[jnp.full((_LANES,), t, jnp.int32)].get(
                mode="promise_in_bounds")
            run = (rl_b - 1) if run is None else run + (rl_b - 1)
            thr.append(run)

        iota = lax.iota(jnp.int32, _LANES)
        lane_i = iota >> 1
        par = iota & 1
        base0 = lane_i + par + (bsz - 1)
        base1 = lane_i + (1 - par) + (bsz - 1)

        e0 = wid * ch

        def body(r, carry):
            g = (e0 + r * _LANES) >> 1
            i_vec = lane_i + g
            b_rel = ((i_vec - thr[0]) >> 31)
            for t in thr[1:]:
                b_rel = b_rel + ((i_vec - t) >> 31)
            b_rel = b_rel + g
            buf0[pl.ds(r * _LANES, _LANES)] = base0 + b_rel
            buf1[pl.ds(r * _LANES, _LANES)] = base1 + b_rel
            return carry

        lax.fori_loop(0, rows, body, 0, unroll=8)

        @pl.when(wid < nw - 1)
        def _():
            pltpu.sync_copy(buf0, c0_hbm.at[pl.ds(wid * ch, ch)])
            pltpu.sync_copy(buf1, c1_hbm.at[pl.ds(wid * ch, ch)])

        @pl.when(wid == nw - 1)
        def _():
            pltpu.sync_copy(buf0.at[pl.ds(0, last)],
                            c0_hbm.at[pl.ds((nw - 1) * ch, last)])
            pltpu.sync_copy(buf1.at[pl.ds(0, last)],
                            c1_hbm.at[pl.ds((nw - 1) * ch, last)])

    return k


@functools.lru_cache(maxsize=None)
def _make_merged_copy(total: int, d: int, nchunks: int = 16, nbuf: int = 4):
    assert total % nchunks == 0
    blk = total // nchunks

    def body(x_hbm, o_hbm, bufs, insems, outsems):
        def in_cp(i):
            return pltpu.make_async_copy(
                x_hbm.at[pl.ds(i * blk, blk)], bufs.at[i % nbuf],
                insems.at[i % nbuf])

        def out_cp(i):
            return pltpu.make_async_copy(
                bufs.at[i % nbuf], o_hbm.at[pl.ds(i * blk, blk)],
                outsems.at[i % nbuf])

        for j in range(min(nbuf, nchunks)):
            in_cp(j).start()
        for i in range(nchunks):
            in_cp(i).wait()
            out_cp(i).start()
            if i + nbuf < nchunks:
                out_cp(i).wait()
                in_cp(i + nbuf).start()
        for i in range(max(0, nchunks - nbuf), nchunks):
            out_cp(i).wait()

    return pl.pallas_call(
        body,
        in_specs=[pl.BlockSpec(memory_space=pltpu.HBM)],
        out_specs=pl.BlockSpec(memory_space=pltpu.HBM),
        scratch_shapes=[
            pltpu.VMEM((nbuf, blk, d), jnp.float32),
            pltpu.SemaphoreType.DMA((nbuf,)),
            pltpu.SemaphoreType.DMA((nbuf,)),
        ],
        out_shape=jax.ShapeDtypeStruct((total, d), jnp.float32),
    )


def kernel(flat, row_lengths):
    total = flat.shape[0]
    bsz = row_lengths.shape[0]
    col0, col1 = _make_edge_cols_kernel(total, bsz)(row_lengths)
    linkages = jnp.stack([col0, col1], axis=1)
    merged = _make_merged_copy(total, flat.shape[1], 8, 8)(flat)
    return merged, linkages


# speedup vs baseline: 1.0120x; 1.0120x over previous
import functools

import jax
import jax.numpy as jnp
from jax import lax
from jax.experimental import pallas as pl
from jax.experimental.pallas import tpu as pltpu
from jax.experimental.pallas import tpu_sc as plsc

_LANES = 16


@functools.lru_cache(maxsize=None)
def _make_edge_cols_kernel(total: int, bsz: int):
    info = plsc.get_sparse_core_info()
    nw = info.num_cores * info.num_subcores
    n_edges = 2 * (total - bsz)
    assert n_edges % _LANES == 0
    ch = -(-n_edges // nw)
    ch = -(-ch // _LANES) * _LANES
    last = n_edges - (nw - 1) * ch
    assert 0 < last <= ch and last % _LANES == 0 and ch % 8 == 0
    rows = ch // _LANES

    mesh = plsc.VectorSubcoreMesh(core_axis_name="c", subcore_axis_name="s")

    @functools.partial(
        pl.kernel,
        mesh=mesh,
        out_type=(
            jax.ShapeDtypeStruct((n_edges,), jnp.int32),
            jax.ShapeDtypeStruct((n_edges,), jnp.int32),
        ),
        scratch_types=[
            pltpu.VMEM((_LANES,), jnp.int32),
            pltpu.VMEM((ch,), jnp.int32),
            pltpu.VMEM((ch,), jnp.int32),
        ],
    )
    def k(rl_hbm, c0_hbm, c1_hbm, rl_v, buf0, buf1):
        wid = lax.axis_index("s") * info.num_cores + lax.axis_index("c")

        pltpu.sync_copy(rl_hbm, rl_v)
        rl_vec = rl_v[...]
        thr = []
        run = None
        for t in range(bsz - 1):
            rl_b = rl_vec.at[jnp.full((_LANES,), t, jnp.int32)].get(
                mode="promise_in_bounds")
            run = (rl_b - 1) if run is None else run + (rl_b - 1)
            thr.append(run)

        iota = lax.iota(jnp.int32, _LANES)
        lane_i = iota >> 1
        par = iota & 1
        base0 = lane_i + par + (bsz - 1)
        base1 = lane_i + (1 - par) + (bsz - 1)

        e0 = wid * ch

        def body(r, carry):
            g = (e0 + r * _LANES) >> 1
            i_vec = lane_i + g
            b_rel = ((i_vec - thr[0]) >> 31)
            for t in thr[1:]:
                b_rel = b_rel + ((i_vec - t) >> 31)
            b_rel = b_rel + g
            buf0[pl.ds(r * _LANES, _LANES)] = base0 + b_rel
            buf1[pl.ds(r * _LANES, _LANES)] = base1 + b_rel
            return carry

        lax.fori_loop(0, rows, body, 0, unroll=2)

        @pl.when(wid < nw - 1)
        def _():
            pltpu.sync_copy(buf0, c0_hbm.at[pl.ds(wid * ch, ch)])
            pltpu.sync_copy(buf1, c1_hbm.at[pl.ds(wid * ch, ch)])

        @pl.when(wid == nw - 1)
        def _():
            pltpu.sync_copy(buf0.at[pl.ds(0, last)],
                            c0_hbm.at[pl.ds((nw - 1) * ch, last)])
            pltpu.sync_copy(buf1.at[pl.ds(0, last)],
                            c1_hbm.at[pl.ds((nw - 1) * ch, last)])

    return k


@functools.lru_cache(maxsize=None)
def _make_merged_copy(total: int, d: int, nchunks: int = 16, nbuf: int = 4):
    assert total % nchunks == 0
    blk = total // nchunks

    def body(x_hbm, o_hbm, bufs, insems, outsems):
        def in_cp(i):
            return pltpu.make_async_copy(
                x_hbm.at[pl.ds(i * blk, blk)], bufs.at[i % nbuf],
                insems.at[i % nbuf])

        def out_cp(i):
            return pltpu.make_async_copy(
                bufs.at[i % nbuf], o_hbm.at[pl.ds(i * blk, blk)],
                outsems.at[i % nbuf])

        for j in range(min(nbuf, nchunks)):
            in_cp(j).start()
        for i in range(nchunks):
            in_cp(i).wait()
            out_cp(i).start()
            if i + nbuf < nchunks:
                out_cp(i).wait()
                in_cp(i + nbuf).start()
        for i in range(max(0, nchunks - nbuf), nchunks):
            out_cp(i).wait()

    return pl.pallas_call(
        body,
        in_specs=[pl.BlockSpec(memory_space=pltpu.HBM)],
        out_specs=pl.BlockSpec(memory_space=pltpu.HBM),
        scratch_shapes=[
            pltpu.VMEM((nbuf, blk, d), jnp.float32),
            pltpu.SemaphoreType.DMA((nbuf,)),
            pltpu.SemaphoreType.DMA((nbuf,)),
        ],
        out_shape=jax.ShapeDtypeStruct((total, d), jnp.float32),
    )


def kernel(flat, row_lengths):
    total = flat.shape[0]
    bsz = row_lengths.shape[0]
    col0, col1 = _make_edge_cols_kernel(total, bsz)(row_lengths)
    linkages = jnp.stack([col0, col1], axis=1)
    merged = _make_merged_copy(total, flat.shape[1], 8, 8)(flat)
    return merged, linkages


# speedup vs baseline: 1.0648x; 1.0522x over previous
import functools

import jax
import jax.numpy as jnp
from jax import lax
from jax.experimental import pallas as pl
from jax.experimental.pallas import tpu as pltpu
from jax.experimental.pallas import tpu_sc as plsc

_LANES = 16


@functools.lru_cache(maxsize=None)
def _make_edge_cols_kernel(total: int, bsz: int):
    info = plsc.get_sparse_core_info()
    nw = 1 * info.num_subcores
    n_edges = 2 * (total - bsz)
    assert n_edges % _LANES == 0
    ch = -(-n_edges // nw)
    ch = -(-ch // _LANES) * _LANES
    last = n_edges - (nw - 1) * ch
    assert 0 < last <= ch and last % _LANES == 0 and ch % 8 == 0
    rows = ch // _LANES

    mesh = plsc.VectorSubcoreMesh(core_axis_name="c", subcore_axis_name="s", num_cores=1)

    @functools.partial(
        pl.kernel,
        mesh=mesh,
        out_type=(
            jax.ShapeDtypeStruct((n_edges,), jnp.int32),
            jax.ShapeDtypeStruct((n_edges,), jnp.int32),
        ),
        scratch_types=[
            pltpu.VMEM((_LANES,), jnp.int32),
            pltpu.VMEM((ch,), jnp.int32),
            pltpu.VMEM((ch,), jnp.int32),
        ],
    )
    def k(rl_hbm, c0_hbm, c1_hbm, rl_v, buf0, buf1):
        wid = lax.axis_index("s")

        pltpu.sync_copy(rl_hbm, rl_v)
        rl_vec = rl_v[...]
        thr = []
        run = None
        for t in range(bsz - 1):
            rl_b = rl_vec.at[jnp.full((_LANES,), t, jnp.int32)].get(
                mode="promise_in_bounds")
            run = (rl_b - 1) if run is None else run + (rl_b - 1)
            thr.append(run)

        iota = lax.iota(jnp.int32, _LANES)
        lane_i = iota >> 1
        par = iota & 1
        base0 = lane_i + par + (bsz - 1)
        base1 = lane_i + (1 - par) + (bsz - 1)

        e0 = wid * ch

        def body(r, carry):
            g = (e0 + r * _LANES) >> 1
            i_vec = lane_i + g
            b_rel = ((i_vec - thr[0]) >> 31)
            for t in thr[1:]:
                b_rel = b_rel + ((i_vec - t) >> 31)
            b_rel = b_rel + g
            buf0[pl.ds(r * _LANES, _LANES)] = base0 + b_rel
            buf1[pl.ds(r * _LANES, _LANES)] = base1 + b_rel
            return carry

        lax.fori_loop(0, rows, body, 0, unroll=2)

        @pl.when(wid < nw - 1)
        def _():
            pltpu.sync_copy(buf0, c0_hbm.at[pl.ds(wid * ch, ch)])
            pltpu.sync_copy(buf1, c1_hbm.at[pl.ds(wid * ch, ch)])

        @pl.when(wid == nw - 1)
        def _():
            pltpu.sync_copy(buf0.at[pl.ds(0, last)],
                            c0_hbm.at[pl.ds((nw - 1) * ch, last)])
            pltpu.sync_copy(buf1.at[pl.ds(0, last)],
                            c1_hbm.at[pl.ds((nw - 1) * ch, last)])

    return k


@functools.lru_cache(maxsize=None)
def _make_merged_copy(total: int, d: int, nchunks: int = 16, nbuf: int = 4):
    assert total % nchunks == 0
    blk = total // nchunks

    def body(x_hbm, o_hbm, bufs, insems, outsems):
        def in_cp(i):
            return pltpu.make_async_copy(
                x_hbm.at[pl.ds(i * blk, blk)], bufs.at[i % nbuf],
                insems.at[i % nbuf])

        def out_cp(i):
            return pltpu.make_async_copy(
                bufs.at[i % nbuf], o_hbm.at[pl.ds(i * blk, blk)],
                outsems.at[i % nbuf])

        for j in range(min(nbuf, nchunks)):
            in_cp(j).start()
        for i in range(nchunks):
            in_cp(i).wait()
            out_cp(i).start()
            if i + nbuf < nchunks:
                out_cp(i).wait()
                in_cp(i + nbuf).start()
        for i in range(max(0, nchunks - nbuf), nchunks):
            out_cp(i).wait()

    return pl.pallas_call(
        body,
        in_specs=[pl.BlockSpec(memory_space=pltpu.HBM)],
        out_specs=pl.BlockSpec(memory_space=pltpu.HBM),
        scratch_shapes=[
            pltpu.VMEM((nbuf, blk, d), jnp.float32),
            pltpu.SemaphoreType.DMA((nbuf,)),
            pltpu.SemaphoreType.DMA((nbuf,)),
        ],
        out_shape=jax.ShapeDtypeStruct((total, d), jnp.float32),
    )


def kernel(flat, row_lengths):
    total = flat.shape[0]
    bsz = row_lengths.shape[0]
    col0, col1 = _make_edge_cols_kernel(total, bsz)(row_lengths)
    linkages = jnp.stack([col0, col1], axis=1)
    merged = _make_merged_copy(total, flat.shape[1], 8, 8)(flat)
    return merged, linkages
